# Initial kernel scaffold; baseline (speedup 1.0000x reference)
import jax
import jax.numpy as jnp
from jax.experimental import pallas as pl


def kernel(x, edge_index, W_in, b_in, W1, as1, ad1, bs1, g1, be1, W2, as2, ad2, bs2, g2, be2, W3, as3, ad3, bs3, g3, be3, Wp, bp, Wc1, bc1, Wc2, bc2, Wc3, bc3, Wt1, bt1, Wt2, bt2, Wt3, bt3, Wf1, bf1, Wf2, bf2):
    raise NotImplementedError("write your pallas kernel here")


# speedup vs baseline: 1.0050x; 1.0050x over previous
import functools

import jax
import jax.numpy as jnp
from jax.experimental import pallas as pl
from jax.experimental.pallas import tpu as pltpu


def _matmul_act(x, w, b=None, act=None, bm=2000):
    M, K = x.shape
    N = w.shape[1]
    if M % bm != 0:
        bm = M
    grid = (M // bm,)

    def body(x_ref, w_ref, b_ref, o_ref):
        acc = jnp.dot(x_ref[...], w_ref[...], preferred_element_type=jnp.float32)
        if b_ref is not None:
            acc = acc + b_ref[...]
        if act == "relu":
            acc = jnp.maximum(acc, 0.0)
        o_ref[...] = acc

    in_specs = [
        pl.BlockSpec((bm, K), lambda i: (i, 0)),
        pl.BlockSpec((K, N), lambda i: (0, 0)),
    ]
    args = [x, w]
    if b is not None:
        in_specs.append(pl.BlockSpec((1, N), lambda i: (0, 0)))
        args.append(b.reshape(1, N))
        f = body
    else:
        f = lambda x_ref, w_ref, o_ref: body(x_ref, w_ref, None, o_ref)
    return pl.pallas_call(
        f,
        grid=grid,
        in_specs=in_specs,
        out_specs=pl.BlockSpec((bm, N), lambda i: (i, 0)),
        out_shape=jax.ShapeDtypeStruct((M, N), jnp.float32),
    )(*args)


def _gat_edge_jnp(hw, al_s, al_d, src, dst, heads, ch, N):
    e = jax.nn.leaky_relu(al_s[src] + al_d[dst], negative_slope=0.2)
    emax = jax.ops.segment_max(e, dst, num_segments=N)
    emax = jnp.where(jnp.isfinite(emax), emax, 0.0)
    ex = jnp.exp(e - emax[dst])
    den = jax.ops.segment_sum(ex, dst, num_segments=N)
    alpha = ex / (den[dst] + 1e-16)
    h3 = hw.reshape(-1, heads, ch)
    out = jax.ops.segment_sum(h3[src] * alpha[:, :, None], dst, num_segments=N)
    return out


def _bn_relu(x, g, b):
    return jnp.maximum(g * (x / jnp.sqrt(1.0 + 1e-5)) + b, 0.0)


def _gat_layer(h, src, dst, W, a_s, a_d, bias, g, be, heads, ch, concat, N):
    hw = _matmul_act(h, W)
    h3 = hw.reshape(N, heads, ch)
    al_s = jnp.sum(h3 * a_s[None, :, :], axis=-1)
    al_d = jnp.sum(h3 * a_d[None, :, :], axis=-1)
    out = _gat_edge_jnp(hw, al_s, al_d, src, dst, heads, ch, N)
    if concat:
        out = out.reshape(N, heads * ch)
    else:
        out = jnp.mean(out, axis=1)
    return _bn_relu(out + bias, g, be)


def kernel(x, edge_index, W_in, b_in, W1, as1, ad1, bs1, g1, be1, W2, as2, ad2, bs2, g2, be2, W3, as3, ad3, bs3, g3, be3, Wp, bp, Wc1, bc1, Wc2, bc2, Wc3, bc3, Wt1, bt1, Wt2, bt2, Wt3, bt3, Wf1, bf1, Wf2, bf2):
    N = x.shape[0]
    loop = jnp.arange(N, dtype=edge_index.dtype)
    src = jnp.concatenate([edge_index[0], loop])
    dst = jnp.concatenate([edge_index[1], loop])

    h = _matmul_act(x, W_in, b_in, act="relu")
    x1 = _gat_layer(h, src, dst, W1, as1, ad1, bs1, g1, be1, 8, 128, True, N)
    x2 = _gat_layer(x1, src, dst, W2, as2, ad2, bs2, g2, be2, 8, 128, True, N)
    x3 = _gat_layer(x2, src, dst, W3, as3, ad3, bs3, g3, be3, 4, 256, False, N)

    pooled = jnp.concatenate(
        [jnp.mean(x3, axis=0), jnp.max(x3, axis=0), jnp.sum(x3, axis=0)]
    )[None, :]
    g = jax.nn.relu(pooled @ Wp + bp)
    vuln = jax.nn.relu(g @ Wc1 + bc1)
    vuln = jax.nn.relu(vuln @ Wc2 + bc2) @ Wc3 + bc3
    vtype = jax.nn.relu(g @ Wt1 + bt1)
    vtype = jax.nn.relu(vtype @ Wt2 + bt2) @ Wt3 + bt3
    conf = jax.nn.sigmoid(jax.nn.relu(g @ Wf1 + bf1) @ Wf2 + bf2)
    return (vuln, vtype, conf)
